# baseline (speedup 1.0000x reference)
import jax
import jax.numpy as jnp
from jax.experimental import pallas as pl
from jax.experimental.pallas import tpu as pltpu

B = 4096
D = 128
DH = 64
C = 101
CP = 128
GAMM = 1.0
ALPH = 0.01
TI = 1024
NB = B // TI
OFFS = tuple(i * NB - i * (i - 1) // 2 for i in range(NB))
NT = NB * (NB + 1) // 2
PK = 5
NSTEP = NT // PK
BF = jnp.bfloat16
F32 = jnp.float32


def _ij(t):
    i = jnp.int32(0)
    for k in range(1, NB):
        i = i + (t >= OFFS[k]).astype(jnp.int32)
    j = t - (i * NB - i * (i - 1) // 2) + i
    return i, j


def _dot_nt(a, b):
    return jax.lax.dot_general(a, b, (((1,), (1,)), ((), ())),
                               preferred_element_type=F32)


def _dot_tn(a, b):
    return jax.lax.dot_general(a, b, (((0,), (0,)), ((), ())),
                               preferred_element_type=F32)


def _dot(a, b):
    return jax.lax.dot_general(a, b, (((1,), (0,)), ((), ())),
                               preferred_element_type=F32)


def _softplus(x):
    u = jnp.exp2(jnp.abs(x) * x.dtype.type(-1.4426950408889634))
    return jnp.maximum(x, x.dtype.type(0)) + jnp.log1p(u)


def _onehot(t, dtype):
    return (t[:, None] ==
            jax.lax.broadcasted_iota(jnp.int32, (TI, CP), 1)).astype(dtype)


def _body(*refs):
    (ho_i, ho_j, hf, cls) = (refs[0:PK], refs[PK:2 * PK],
                             refs[2 * PK:3 * PK], refs[3 * PK:4 * PK])
    (tg_i, tg_j) = (refs[4 * PK:5 * PK], refs[5 * PK:6 * PK])
    (ho6s, ho6b, tg0, simL, simn, cntm,
     sim_out, cnt_out, stats_out,
     accG, accSh, accNC, accU, accDV,
     s_SP, s_sp, s_dPsp, s_dPth, s_nll) = refs[6 * PK:]
    t = pl.program_id(0)

    @pl.when(t == 0)
    def _init():
        accG[:, :] = jnp.zeros((CP, D), F32)
        accSh[:, :] = jnp.zeros((CP, DH), F32)
        for r in (accNC, accU, accDV):
            r[:, :] = jnp.zeros((1, CP), F32)
        for r in (s_SP, s_sp, s_dPsp, s_dPth, s_nll):
            r[:, :] = jnp.zeros((1, 1), F32)

    tis = [tg_i[k][0, 0, :] for k in range(PK)]
    tjs = [tg_j[k][0, 0, :] for k in range(PK)]
    ohis = [_onehot(ti, F32) for ti in tis]
    ohjs = [_onehot(tj, BF) for tj in tjs]
    thetas = [_dot_nt(ho_i[k][:, :], ho_j[k][:, :]) for k in range(PK)]
    sps = [_softplus(th.astype(BF)) for th in thetas]
    tmps = [_dot(sp, ohj) for sp, ohj in zip(sps, ohjs)]
    ws = []
    for k in range(PK):
        i, j = _ij(PK * t + k)
        ws.append(jnp.where(i == j, 1.0, 2.0))
    d_SP = sum(w * jnp.sum(ohi * tmp) for w, ohi, tmp in zip(ws, ohis, tmps))
    d_sp = sum(w * jnp.sum(tmp) for w, tmp in zip(ws, tmps))
    s_SP[:, :] = s_SP[:, :] + d_SP.reshape(1, 1)
    s_sp[:, :] = s_sp[:, :] + d_sp.reshape(1, 1)

    def _row0(cls_j, hf_j, ho_j, ohj, tj):
        ohjf = ohj.astype(F32)
        x = cls_j[:, :]
        m = jnp.max(x, axis=1, keepdims=True)
        lane = jax.lax.broadcasted_iota(jnp.int32, (TI, CP), 1)
        firstmax = jnp.min(jnp.where(x == m, lane, CP), axis=1)
        corr = (firstmax == tj).astype(F32)
        lse = m[:, 0] + jnp.log(jnp.sum(jnp.exp(x - m), axis=1))
        logit = jnp.sum(x * ohjf, axis=1)
        s_nll[:, :] = s_nll[:, :] + jnp.sum(lse - logit).reshape(1, 1)
        xf = hf_j[:, :]
        nrm = jnp.sqrt(jnp.sum(xf * xf, axis=1, keepdims=True))
        xn = (xf / jnp.maximum(nrm, 1e-12)).astype(BF)
        ohc = ohjf * corr[:, None]
        accG[:, :] = accG[:, :] + _dot_tn(ohc.astype(BF), xn)
        accSh[:, :] = accSh[:, :] + _dot_tn(ohj, ho_j[:, :])
        accNC[:, :] = accNC[:, :] + jnp.sum(ohjf, axis=0).reshape(1, CP)
        accU[:, :] = accU[:, :] + jnp.sum(ohc, axis=0).reshape(1, CP)
        xnf = xn.astype(F32)
        selfsim = jnp.sum(xnf * xnf, axis=1)
        accDV[:, :] = accDV[:, :] + jnp.sum(ohc * selfsim[:, None],
                                            axis=0).reshape(1, CP)
        hjf = ho_j[:, :].astype(F32)
        thd = 0.5 * jnp.sum(hjf * hjf, axis=1)
        spd = _softplus(thd.astype(BF)).astype(F32)
        s_dPsp[:, :] = s_dPsp[:, :] + jnp.sum(spd).reshape(1, 1)
        s_dPth[:, :] = s_dPth[:, :] + jnp.sum(thd).reshape(1, 1)

    for k in range(PK):
        @pl.when(PK * t + k < NB)
        def _row0_k(k=k):
            _row0(cls[k], hf[k], ho_j[k], ohjs[k], tjs[k])

    @pl.when(t == NSTEP - 1)
    def _final():
        th6 = _dot_nt(ho6s[:, :], ho6b[:, :])
        t8 = tg0[0, 0, :8]
        oh6 = (t8[:, None] ==
               jax.lax.broadcasted_iota(jnp.int32, (8, CP), 1)).astype(F32)
        g6 = _dot_nt(_dot(oh6, simL[:, :]), oh6)
        r8 = jax.lax.broadcasted_iota(jnp.int32, (8, 8), 0)
        c8 = jax.lax.broadcasted_iota(jnp.int32, (8, 8), 1)
        valid = ((r8 < 6) & (c8 < 6)).astype(F32)
        eye8 = (r8 == c8).astype(F32)
        sim_old = (t8[:, None] == t8[None, :]).astype(F32)
        pos_new = (g6 == 1.0).astype(F32)
        sp6 = _softplus(th6.astype(BF)).astype(F32)
        pl_old = sp6 - sim_old * th6
        pl_new = sp6 - g6 * th6
        d_pos_term = (pl_new * pos_new - pl_old * sim_old) * valid
        d_neg_term = (pl_new * (1.0 - pos_new) - pl_old * (1.0 - sim_old)) * valid

        nc = accNC[:, :]
        cntP = jnp.sum(nc * nc) + jnp.sum((pos_new - sim_old) * valid)
        sh = accSh[:, :]
        THt = 0.5 * jnp.sum(sh * sh)
        SPt = jnp.sum(s_SP[:, :])
        SPa = jnp.sum(s_sp[:, :])
        P = SPt - THt + jnp.sum(d_pos_term)
        Nn = SPa - SPt + jnp.sum(d_neg_term)
        dPd = jnp.sum(s_dPsp[:, :]) - jnp.sum(s_dPth[:, :]) \
            + jnp.sum(d_pos_term * eye8)
        dNd = jnp.sum(d_neg_term * eye8)

        Bf = jnp.float32(B)
        S1 = cntP - Bf
        S0 = Bf * Bf - cntP
        S0 = jnp.where(S0 == 0.0, 1.0, S0)
        S1 = jnp.where(S1 == 0.0, 1.0, S1)
        S = S0 + S1
        total = (P - dPd) * (S / S1) + (Nn - dNd) * (S / S0)
        hash_loss = total / 2.0 / (Bf * (Bf - 1.0) / 2.0)
        cls_loss = jnp.sum(s_nll[:, :]) / Bf
        loss = GAMM * cls_loss + ALPH * hash_loss

        eyeC = (jax.lax.broadcasted_iota(jnp.int32, (CP, CP), 0) ==
                jax.lax.broadcasted_iota(jnp.int32, (CP, CP), 1))
        G = accG[:, :]
        FA = _dot_nt(G, G) - jnp.where(
            eyeC, jnp.broadcast_to(accDV[:, :], (CP, CP)), 0.0)
        sim_out[:, :] = simn[:, :] + FA - jnp.where(eyeC, FA, 0.0) * 0.5
        U = accU[:, :]
        FC = U.T * U - jnp.where(eyeC, jnp.broadcast_to(U, (CP, CP)), 0.0)
        cnt_out[:, :] = cntm[:, :] + FC - jnp.where(eyeC, FC, 0.0) * 0.5
        lane = jax.lax.broadcasted_iota(jnp.int32, (1, CP), 1)
        stats_out[:, :] = (hash_loss * (lane == 0) + cls_loss * (lane == 1)
                           + loss * (lane == 2)).astype(F32)


def kernel(hash_feature, hash_out, cls_out, target, sim_matrix_last,
           sim_matrix_now, count_matrix, epoch):
    del epoch
    hos = (hash_out * 0.5).astype(BF)
    hob = hash_out.astype(BF)
    cls_pad = jnp.pad(cls_out, ((0, 0), (0, CP - C)), constant_values=-1e30)
    simL_pad = jnp.pad(sim_matrix_last, ((0, CP - C), (0, CP - C)))
    simn_pad = jnp.pad(sim_matrix_now, ((0, CP - C), (0, CP - C)))
    cnt_pad = jnp.pad(count_matrix, ((0, CP - C), (0, CP - C)))
    tgt3 = target.astype(jnp.int32).reshape(NB, 1, TI)

    def mk_i(k):
        def im(t):
            i, _ = _ij(PK * t + k)
            return (i, 0)
        return im

    def mk_j(k):
        def im(t):
            _, j = _ij(PK * t + k)
            return (j, 0)
        return im

    def mk_j0(k):
        def im(t):
            _, j = _ij(PK * t + k)
            return (jnp.where(PK * t + k < NB, j, 0), 0)
        return im

    def mk_ti(k):
        def im(t):
            i, _ = _ij(PK * t + k)
            return (i, 0, 0)
        return im

    def mk_tj(k):
        def im(t):
            _, j = _ij(PK * t + k)
            return (j, 0, 0)
        return im

    const2 = lambda t: (0, 0)
    const3 = lambda t: (0, 0, 0)

    in_specs = (
        [pl.BlockSpec((TI, DH), mk_i(k)) for k in range(PK)] +
        [pl.BlockSpec((TI, DH), mk_j(k)) for k in range(PK)] +
        [pl.BlockSpec((TI, D), mk_j0(k)) for k in range(PK)] +
        [pl.BlockSpec((TI, CP), mk_j0(k)) for k in range(PK)] +
        [pl.BlockSpec((1, 1, TI), mk_ti(k)) for k in range(PK)] +
        [pl.BlockSpec((1, 1, TI), mk_tj(k)) for k in range(PK)] +
        [
            pl.BlockSpec((8, DH), const2),
            pl.BlockSpec((8, DH), const2),
            pl.BlockSpec((1, 1, TI), const3),
            pl.BlockSpec((CP, CP), const2),
            pl.BlockSpec((CP, CP), const2),
            pl.BlockSpec((CP, CP), const2),
        ])

    grid_spec = pltpu.PrefetchScalarGridSpec(
        num_scalar_prefetch=0,
        grid=(NSTEP,),
        in_specs=in_specs,
        out_specs=[
            pl.BlockSpec((CP, CP), const2),
            pl.BlockSpec((CP, CP), const2),
            pl.BlockSpec((1, CP), const2),
        ],
        scratch_shapes=[
            pltpu.VMEM((CP, D), F32),
            pltpu.VMEM((CP, DH), F32),
            pltpu.VMEM((1, CP), F32),
            pltpu.VMEM((1, CP), F32),
            pltpu.VMEM((1, CP), F32),
        ] + [pltpu.VMEM((1, 1), F32) for _ in range(5)],
    )

    sim_p, cnt_p, stats = pl.pallas_call(
        _body,
        grid_spec=grid_spec,
        out_shape=[
            jax.ShapeDtypeStruct((CP, CP), F32),
            jax.ShapeDtypeStruct((CP, CP), F32),
            jax.ShapeDtypeStruct((1, CP), F32),
        ],
    )(*([hos] * PK + [hob] * PK + [hash_feature] * PK + [cls_pad] * PK
        + [tgt3] * PK + [tgt3] * PK
        + [hos, hob, tgt3, simL_pad, simn_pad, cnt_pad]))

    return (sim_p[:C, :C], cnt_p[:C, :C],
            stats[0, 0], stats[0, 1], stats[0, 2])


# speedup vs baseline: 1.0975x; 1.0975x over previous
import jax
import jax.numpy as jnp
from jax.experimental import pallas as pl
from jax.experimental.pallas import tpu as pltpu

B = 4096
D = 128
DH = 64
C = 101
CP = 128
GAMM = 1.0
ALPH = 0.01
TI = 2048
NB = B // TI
OFFS = tuple(i * NB - i * (i - 1) // 2 for i in range(NB))
NT = NB * (NB + 1) // 2
PK = 1
NSTEP = NT // PK
BF = jnp.bfloat16
F32 = jnp.float32


def _ij(t):
    i = jnp.int32(0)
    for k in range(1, NB):
        i = i + (t >= OFFS[k]).astype(jnp.int32)
    j = t - (i * NB - i * (i - 1) // 2) + i
    return i, j


def _dot_nt(a, b):
    return jax.lax.dot_general(a, b, (((1,), (1,)), ((), ())),
                               preferred_element_type=F32)


def _dot_tn(a, b):
    return jax.lax.dot_general(a, b, (((0,), (0,)), ((), ())),
                               preferred_element_type=F32)


def _dot(a, b):
    return jax.lax.dot_general(a, b, (((1,), (0,)), ((), ())),
                               preferred_element_type=F32)


def _softplus(x):
    u = jnp.exp2(jnp.abs(x) * x.dtype.type(-1.4426950408889634))
    return jnp.maximum(x, x.dtype.type(0)) + jnp.log1p(u)


def _onehot(t, dtype):
    return (t[:, None] ==
            jax.lax.broadcasted_iota(jnp.int32, (TI, CP), 1)).astype(dtype)


def _body(*refs):
    (ho_i, ho_j, hf, cls) = (refs[0:PK], refs[PK:2 * PK],
                             refs[2 * PK:3 * PK], refs[3 * PK:4 * PK])
    (tg_i, tg_j) = (refs[4 * PK:5 * PK], refs[5 * PK:6 * PK])
    (ho6s, ho6b, tg0, simL, simn, cntm,
     sim_out, cnt_out, stats_out,
     accG, accSh, accNC, accU, accDV,
     s_SP, s_sp, s_dPsp, s_dPth, s_nll) = refs[6 * PK:]
    t = pl.program_id(0)

    @pl.when(t == 0)
    def _init():
        accG[:, :] = jnp.zeros((CP, D), F32)
        accSh[:, :] = jnp.zeros((CP, DH), F32)
        for r in (accNC, accU, accDV):
            r[:, :] = jnp.zeros((1, CP), F32)
        for r in (s_SP, s_sp, s_dPsp, s_dPth, s_nll):
            r[:, :] = jnp.zeros((1, 1), F32)

    tis = [tg_i[k][0, 0, :] for k in range(PK)]
    tjs = [tg_j[k][0, 0, :] for k in range(PK)]
    ohis = [_onehot(ti, F32) for ti in tis]
    ohjs = [_onehot(tj, BF) for tj in tjs]
    thetas = [_dot_nt(ho_i[k][:, :], ho_j[k][:, :]) for k in range(PK)]
    sps = [_softplus(th.astype(BF)) for th in thetas]
    tmps = [_dot(sp, ohj) for sp, ohj in zip(sps, ohjs)]
    ws = []
    for k in range(PK):
        i, j = _ij(PK * t + k)
        ws.append(jnp.where(i == j, 1.0, 2.0))
    d_SP = sum(w * jnp.sum(ohi * tmp) for w, ohi, tmp in zip(ws, ohis, tmps))
    d_sp = sum(w * jnp.sum(tmp) for w, tmp in zip(ws, tmps))
    s_SP[:, :] = s_SP[:, :] + d_SP.reshape(1, 1)
    s_sp[:, :] = s_sp[:, :] + d_sp.reshape(1, 1)

    def _row0(cls_j, hf_j, ho_j, ohj, tj):
        ohjf = ohj.astype(F32)
        x = cls_j[:, :]
        m = jnp.max(x, axis=1, keepdims=True)
        lane = jax.lax.broadcasted_iota(jnp.int32, (TI, CP), 1)
        firstmax = jnp.min(jnp.where(x == m, lane, CP), axis=1)
        corr = (firstmax == tj).astype(F32)
        lse = m[:, 0] + jnp.log(jnp.sum(jnp.exp(x - m), axis=1))
        logit = jnp.sum(x * ohjf, axis=1)
        s_nll[:, :] = s_nll[:, :] + jnp.sum(lse - logit).reshape(1, 1)
        xf = hf_j[:, :]
        nrm = jnp.sqrt(jnp.sum(xf * xf, axis=1, keepdims=True))
        xn = (xf / jnp.maximum(nrm, 1e-12)).astype(BF)
        ohc = ohjf * corr[:, None]
        accG[:, :] = accG[:, :] + _dot_tn(ohc.astype(BF), xn)
        accSh[:, :] = accSh[:, :] + _dot_tn(ohj, ho_j[:, :])
        accNC[:, :] = accNC[:, :] + jnp.sum(ohjf, axis=0).reshape(1, CP)
        accU[:, :] = accU[:, :] + jnp.sum(ohc, axis=0).reshape(1, CP)
        xnf = xn.astype(F32)
        selfsim = jnp.sum(xnf * xnf, axis=1)
        accDV[:, :] = accDV[:, :] + jnp.sum(ohc * selfsim[:, None],
                                            axis=0).reshape(1, CP)
        hjf = ho_j[:, :].astype(F32)
        thd = 0.5 * jnp.sum(hjf * hjf, axis=1)
        spd = _softplus(thd.astype(BF)).astype(F32)
        s_dPsp[:, :] = s_dPsp[:, :] + jnp.sum(spd).reshape(1, 1)
        s_dPth[:, :] = s_dPth[:, :] + jnp.sum(thd).reshape(1, 1)

    for k in range(PK):
        @pl.when(PK * t + k < NB)
        def _row0_k(k=k):
            _row0(cls[k], hf[k], ho_j[k], ohjs[k], tjs[k])

    @pl.when(t == NSTEP - 1)
    def _final():
        th6 = _dot_nt(ho6s[:, :], ho6b[:, :])
        t8 = tg0[0, 0, :8]
        oh6 = (t8[:, None] ==
               jax.lax.broadcasted_iota(jnp.int32, (8, CP), 1)).astype(F32)
        g6 = _dot_nt(_dot(oh6, simL[:, :]), oh6)
        r8 = jax.lax.broadcasted_iota(jnp.int32, (8, 8), 0)
        c8 = jax.lax.broadcasted_iota(jnp.int32, (8, 8), 1)
        valid = ((r8 < 6) & (c8 < 6)).astype(F32)
        eye8 = (r8 == c8).astype(F32)
        sim_old = (t8[:, None] == t8[None, :]).astype(F32)
        pos_new = (g6 == 1.0).astype(F32)
        sp6 = _softplus(th6.astype(BF)).astype(F32)
        pl_old = sp6 - sim_old * th6
        pl_new = sp6 - g6 * th6
        d_pos_term = (pl_new * pos_new - pl_old * sim_old) * valid
        d_neg_term = (pl_new * (1.0 - pos_new) - pl_old * (1.0 - sim_old)) * valid

        nc = accNC[:, :]
        cntP = jnp.sum(nc * nc) + jnp.sum((pos_new - sim_old) * valid)
        sh = accSh[:, :]
        THt = 0.5 * jnp.sum(sh * sh)
        SPt = jnp.sum(s_SP[:, :])
        SPa = jnp.sum(s_sp[:, :])
        P = SPt - THt + jnp.sum(d_pos_term)
        Nn = SPa - SPt + jnp.sum(d_neg_term)
        dPd = jnp.sum(s_dPsp[:, :]) - jnp.sum(s_dPth[:, :]) \
            + jnp.sum(d_pos_term * eye8)
        dNd = jnp.sum(d_neg_term * eye8)

        Bf = jnp.float32(B)
        S1 = cntP - Bf
        S0 = Bf * Bf - cntP
        S0 = jnp.where(S0 == 0.0, 1.0, S0)
        S1 = jnp.where(S1 == 0.0, 1.0, S1)
        S = S0 + S1
        total = (P - dPd) * (S / S1) + (Nn - dNd) * (S / S0)
        hash_loss = total / 2.0 / (Bf * (Bf - 1.0) / 2.0)
        cls_loss = jnp.sum(s_nll[:, :]) / Bf
        loss = GAMM * cls_loss + ALPH * hash_loss

        eyeC = (jax.lax.broadcasted_iota(jnp.int32, (CP, CP), 0) ==
                jax.lax.broadcasted_iota(jnp.int32, (CP, CP), 1))
        G = accG[:, :]
        FA = _dot_nt(G, G) - jnp.where(
            eyeC, jnp.broadcast_to(accDV[:, :], (CP, CP)), 0.0)
        sim_out[:, :] = simn[:, :] + FA - jnp.where(eyeC, FA, 0.0) * 0.5
        U = accU[:, :]
        FC = U.T * U - jnp.where(eyeC, jnp.broadcast_to(U, (CP, CP)), 0.0)
        cnt_out[:, :] = cntm[:, :] + FC - jnp.where(eyeC, FC, 0.0) * 0.5
        lane = jax.lax.broadcasted_iota(jnp.int32, (1, CP), 1)
        stats_out[:, :] = (hash_loss * (lane == 0) + cls_loss * (lane == 1)
                           + loss * (lane == 2)).astype(F32)


def kernel(hash_feature, hash_out, cls_out, target, sim_matrix_last,
           sim_matrix_now, count_matrix, epoch):
    del epoch
    hos = (hash_out * 0.5).astype(BF)
    hob = hash_out.astype(BF)
    cls_pad = jnp.pad(cls_out, ((0, 0), (0, CP - C)), constant_values=-1e30)
    simL_pad = jnp.pad(sim_matrix_last, ((0, CP - C), (0, CP - C)))
    simn_pad = jnp.pad(sim_matrix_now, ((0, CP - C), (0, CP - C)))
    cnt_pad = jnp.pad(count_matrix, ((0, CP - C), (0, CP - C)))
    tgt3 = target.astype(jnp.int32).reshape(NB, 1, TI)

    def mk_i(k):
        def im(t):
            i, _ = _ij(PK * t + k)
            return (i, 0)
        return im

    def mk_j(k):
        def im(t):
            _, j = _ij(PK * t + k)
            return (j, 0)
        return im

    def mk_j0(k):
        def im(t):
            _, j = _ij(PK * t + k)
            return (jnp.where(PK * t + k < NB, j, 0), 0)
        return im

    def mk_ti(k):
        def im(t):
            i, _ = _ij(PK * t + k)
            return (i, 0, 0)
        return im

    def mk_tj(k):
        def im(t):
            _, j = _ij(PK * t + k)
            return (j, 0, 0)
        return im

    const2 = lambda t: (0, 0)
    const3 = lambda t: (0, 0, 0)

    in_specs = (
        [pl.BlockSpec((TI, DH), mk_i(k)) for k in range(PK)] +
        [pl.BlockSpec((TI, DH), mk_j(k)) for k in range(PK)] +
        [pl.BlockSpec((TI, D), mk_j0(k)) for k in range(PK)] +
        [pl.BlockSpec((TI, CP), mk_j0(k)) for k in range(PK)] +
        [pl.BlockSpec((1, 1, TI), mk_ti(k)) for k in range(PK)] +
        [pl.BlockSpec((1, 1, TI), mk_tj(k)) for k in range(PK)] +
        [
            pl.BlockSpec((8, DH), const2),
            pl.BlockSpec((8, DH), const2),
            pl.BlockSpec((1, 1, TI), const3),
            pl.BlockSpec((CP, CP), const2),
            pl.BlockSpec((CP, CP), const2),
            pl.BlockSpec((CP, CP), const2),
        ])

    grid_spec = pltpu.PrefetchScalarGridSpec(
        num_scalar_prefetch=0,
        grid=(NSTEP,),
        in_specs=in_specs,
        out_specs=[
            pl.BlockSpec((CP, CP), const2),
            pl.BlockSpec((CP, CP), const2),
            pl.BlockSpec((1, CP), const2),
        ],
        scratch_shapes=[
            pltpu.VMEM((CP, D), F32),
            pltpu.VMEM((CP, DH), F32),
            pltpu.VMEM((1, CP), F32),
            pltpu.VMEM((1, CP), F32),
            pltpu.VMEM((1, CP), F32),
        ] + [pltpu.VMEM((1, 1), F32) for _ in range(5)],
    )

    sim_p, cnt_p, stats = pl.pallas_call(
        _body,
        grid_spec=grid_spec,
        out_shape=[
            jax.ShapeDtypeStruct((CP, CP), F32),
            jax.ShapeDtypeStruct((CP, CP), F32),
            jax.ShapeDtypeStruct((1, CP), F32),
        ],
    )(*([hos] * PK + [hob] * PK + [hash_feature] * PK + [cls_pad] * PK
        + [tgt3] * PK + [tgt3] * PK
        + [hos, hob, tgt3, simL_pad, simn_pad, cnt_pad]))

    return (sim_p[:C, :C], cnt_p[:C, :C],
            stats[0, 0], stats[0, 1], stats[0, 2])
